# baseline (speedup 1.0000x reference)
import functools

import jax
import jax.numpy as jnp
from jax import lax
from jax.experimental import pallas as pl
from jax.experimental.pallas import tpu as pltpu
from jax.experimental.pallas import tpu_sc as plsc

N_NODES = 10000
D = 128
N_EDGES = 320000

NPAD = 10240
NW = 32
TILES = 16
CPT = D // NW
SEGE = 4096
EPAD = 327680
NSEGE = EPAD // SEGE
CW = CPT * NPAD

_mesh = plsc.VectorSubcoreMesh(core_axis_name="c", subcore_axis_name="s")


@functools.partial(
    pl.kernel,
    mesh=_mesh,
    compiler_params=pltpu.CompilerParams(needs_layout_passes=False),
    out_type=jax.ShapeDtypeStruct((D * NPAD,), jnp.float32),
    scratch_types=[
        pltpu.VMEM((CW,), jnp.float32),
        pltpu.VMEM((CW,), jnp.float32),
        pltpu.VMEM((2, SEGE), jnp.int32),
        pltpu.VMEM((2, SEGE), jnp.int32),
        pltpu.SemaphoreType.DMA,
        pltpu.SemaphoreType.DMA,
        pltpu.SemaphoreType.DMA,
        pltpu.SemaphoreType.DMA,
    ],
)
def _msg_call(y_hbm, src_hbm, dst_hbm, out_hbm,
              y_v, acc_v, src_v, dst_v, sems0, sems1, semd0, semd1):
    semss = [sems0, sems1]
    semds = [semd0, semd1]
    c = lax.axis_index("c")
    s = lax.axis_index("s")
    wid = c * TILES + s

    zero16 = jnp.zeros((16,), jnp.float32)

    def zbody(i, carry):
        acc_v[pl.ds(i * 16, 16)] = zero16
        return carry

    lax.fori_loop(0, CW // 16, zbody, 0)

    pltpu.sync_copy(y_hbm.at[pl.ds(wid * CW, CW)], y_v)

    pltpu.sync_copy(src_hbm.at[pl.ds(0, SEGE)], src_v.at[0])
    pltpu.sync_copy(dst_hbm.at[pl.ds(0, SEGE)], dst_v.at[0])
    pltpu.async_copy(src_hbm.at[pl.ds(SEGE, SEGE)], src_v.at[1], sems1)
    pltpu.async_copy(dst_hbm.at[pl.ds(SEGE, SEGE)], dst_v.at[1], semd1)

    colbase = [jnp.full((16,), kk * NPAD, jnp.int32) for kk in range(CPT)]

    def process(k, slot):
        def blk(b, carry):
            s16 = src_v[slot, pl.ds(b * 16, 16)]
            d16 = dst_v[slot, pl.ds(b * 16, 16)]
            for kk in range(CPT):
                v = plsc.load_gather(y_v, [s16 + colbase[kk]])
                plsc.addupdate_scatter(acc_v, [d16 + colbase[kk]], v)
            return carry

        lax.fori_loop(0, SEGE // 16, blk, 0)

        @pl.when(k + 2 < NSEGE)
        def _():
            pltpu.async_copy(src_hbm.at[pl.ds((k + 2) * SEGE, SEGE)],
                             src_v.at[slot], semss[slot])
            pltpu.async_copy(dst_hbm.at[pl.ds((k + 2) * SEGE, SEGE)],
                             dst_v.at[slot], semds[slot])

    def wait_idx(slot):
        pltpu.make_async_copy(src_hbm.at[pl.ds(0, SEGE)],
                              src_v.at[slot], semss[slot]).wait()
        pltpu.make_async_copy(dst_hbm.at[pl.ds(0, SEGE)],
                              dst_v.at[slot], semds[slot]).wait()

    def body(k2, carry):
        ka = 2 * k2

        @pl.when(ka > 0)
        def _():
            wait_idx(0)

        process(ka, 0)
        wait_idx(1)
        process(ka + 1, 1)
        return carry

    lax.fori_loop(0, NSEGE // 2, body, 0)

    pltpu.sync_copy(acc_v, out_hbm.at[pl.ds(wid * CW, CW)])


BR = 1280
DEGW = 16


def _tc1_body(xt_ref, w1t_ref, cnt_ref, y_ref, d_ref):
    d = lax.rsqrt(cnt_ref[0:1, :] + 1.0)
    y_ref[...] = jnp.dot(w1t_ref[...], xt_ref[...],
                         preferred_element_type=jnp.float32) * d
    d_ref[...] = jnp.broadcast_to(d, (DEGW, BR))


def _tc2_body(acc_ref, y1_ref, d_ref, b1_ref, w2t_ref, y2_ref):
    d = d_ref[0:1, :]
    h = jnp.maximum((acc_ref[...] + y1_ref[...]) * d + b1_ref[:, 0:1], 0.0)
    y2_ref[...] = jnp.dot(w2t_ref[...], h,
                          preferred_element_type=jnp.float32) * d


def _tc3_body(acc_ref, y2_ref, d_ref, b2_ref, o_ref):
    d = d_ref[0:1, :]
    o_ref[...] = (acc_ref[...] + y2_ref[...]) * d + b2_ref[:, 0:1]


_tc1 = pl.pallas_call(
    _tc1_body,
    grid=(NPAD // BR,),
    in_specs=[
        pl.BlockSpec((D, BR), lambda i: (0, i)),
        pl.BlockSpec((D, D), lambda i: (0, 0)),
        pl.BlockSpec((1, BR), lambda i: (0, i)),
    ],
    out_specs=[
        pl.BlockSpec((D, BR), lambda i: (0, i)),
        pl.BlockSpec((DEGW, BR), lambda i: (0, i)),
    ],
    out_shape=[
        jax.ShapeDtypeStruct((D, NPAD), jnp.float32),
        jax.ShapeDtypeStruct((DEGW, NPAD), jnp.float32),
    ],
)

_tc2 = pl.pallas_call(
    _tc2_body,
    grid=(NPAD // BR,),
    in_specs=[
        pl.BlockSpec((D, BR), lambda i: (0, i)),
        pl.BlockSpec((D, BR), lambda i: (0, i)),
        pl.BlockSpec((DEGW, BR), lambda i: (0, i)),
        pl.BlockSpec((D, D), lambda i: (0, 0)),
        pl.BlockSpec((D, D), lambda i: (0, 0)),
    ],
    out_specs=pl.BlockSpec((D, BR), lambda i: (0, i)),
    out_shape=jax.ShapeDtypeStruct((D, NPAD), jnp.float32),
)

_tc3 = pl.pallas_call(
    _tc3_body,
    grid=(NPAD // BR,),
    in_specs=[
        pl.BlockSpec((D, BR), lambda i: (0, i)),
        pl.BlockSpec((D, BR), lambda i: (0, i)),
        pl.BlockSpec((DEGW, BR), lambda i: (0, i)),
        pl.BlockSpec((D, D), lambda i: (0, 0)),
    ],
    out_specs=pl.BlockSpec((D, BR), lambda i: (0, i)),
    out_shape=jax.ShapeDtypeStruct((D, NPAD), jnp.float32),
)


@jax.jit
def kernel(x, edge_index, W1, b1, W2, b2):
    src = edge_index[0].astype(jnp.int32)
    dst = edge_index[1].astype(jnp.int32)
    pad = jnp.full((EPAD - N_EDGES,), N_NODES, jnp.int32)
    srcp = jnp.concatenate([src, pad])
    dstp = jnp.concatenate([dst, pad])

    x_pad = jnp.zeros((NPAD, D), jnp.float32).at[:N_NODES].set(x)
    xt = x_pad.T
    b1m = jnp.broadcast_to(b1.reshape(D, 1), (D, D))
    b2m = jnp.broadcast_to(b2.reshape(D, 1), (D, D))
    ones_flat = jnp.ones((D * NPAD,), jnp.float32)

    cnt = _msg_call(ones_flat, srcp, dstp).reshape(D, NPAD)
    y1, dmat = _tc1(xt, W1.T, cnt[0:1])
    acc1 = _msg_call(y1.reshape(-1), srcp, dstp).reshape(D, NPAD)
    y2 = _tc2(acc1, y1, dmat, b1m, W2.T)
    acc2 = _msg_call(y2.reshape(-1), srcp, dstp).reshape(D, NPAD)
    out = _tc3(acc2, y2, dmat, b2m)
    return out.T[:N_NODES]


# speedup vs baseline: 1.4475x; 1.4475x over previous
import functools

import jax
import jax.numpy as jnp
from jax import lax
from jax.experimental import pallas as pl
from jax.experimental.pallas import tpu as pltpu
from jax.experimental.pallas import tpu_sc as plsc

N_NODES = 10000
D = 128
N_EDGES = 320000

NPAD = 10240
NW = 32
TILES = 16
CPT = D // NW
SEGE = 4096
EPAD = 327680
NSEGE = EPAD // SEGE
CW = CPT * NPAD

_mesh = plsc.VectorSubcoreMesh(core_axis_name="c", subcore_axis_name="s")


@functools.partial(
    pl.kernel,
    mesh=_mesh,
    compiler_params=pltpu.CompilerParams(needs_layout_passes=False),
    out_type=jax.ShapeDtypeStruct((D * NPAD,), jnp.float32),
    scratch_types=[
        pltpu.VMEM((CW,), jnp.float32),
        pltpu.VMEM((CW,), jnp.float32),
        pltpu.VMEM((2, SEGE), jnp.int32),
        pltpu.VMEM((2, SEGE), jnp.int32),
        pltpu.SemaphoreType.DMA,
        pltpu.SemaphoreType.DMA,
        pltpu.SemaphoreType.DMA,
        pltpu.SemaphoreType.DMA,
    ],
)
def _msg_call(y_hbm, src_hbm, dst_hbm, out_hbm,
              y_v, acc_v, src_v, dst_v, sems0, sems1, semd0, semd1):
    semss = [sems0, sems1]
    semds = [semd0, semd1]
    c = lax.axis_index("c")
    s = lax.axis_index("s")
    wid = c * TILES + s

    zero16 = jnp.zeros((16,), jnp.float32)

    def zbody(i, carry):
        acc_v[pl.ds(i * 16, 16)] = zero16
        return carry

    lax.fori_loop(0, CW // 16, zbody, 0)

    pltpu.sync_copy(y_hbm.at[pl.ds(wid * CW, CW)], y_v)

    pltpu.sync_copy(src_hbm.at[pl.ds(0, SEGE)], src_v.at[0])
    pltpu.sync_copy(dst_hbm.at[pl.ds(0, SEGE)], dst_v.at[0])
    pltpu.async_copy(src_hbm.at[pl.ds(SEGE, SEGE)], src_v.at[1], sems1)
    pltpu.async_copy(dst_hbm.at[pl.ds(SEGE, SEGE)], dst_v.at[1], semd1)

    colbase = [jnp.full((16,), kk * NPAD, jnp.int32) for kk in range(CPT)]

    def process(k, slot):
        def blk(b, carry):
            s16 = src_v[slot, pl.ds(b * 16, 16)]
            d16 = dst_v[slot, pl.ds(b * 16, 16)]
            for kk in range(CPT):
                v = plsc.load_gather(y_v, [s16 + colbase[kk]])
                plsc.addupdate_scatter(acc_v, [d16 + colbase[kk]], v)
            return carry

        lax.fori_loop(0, SEGE // 16, blk, 0)

        @pl.when(k + 2 < NSEGE)
        def _():
            pltpu.async_copy(src_hbm.at[pl.ds((k + 2) * SEGE, SEGE)],
                             src_v.at[slot], semss[slot])
            pltpu.async_copy(dst_hbm.at[pl.ds((k + 2) * SEGE, SEGE)],
                             dst_v.at[slot], semds[slot])

    def wait_idx(slot):
        pltpu.make_async_copy(src_hbm.at[pl.ds(0, SEGE)],
                              src_v.at[slot], semss[slot]).wait()
        pltpu.make_async_copy(dst_hbm.at[pl.ds(0, SEGE)],
                              dst_v.at[slot], semds[slot]).wait()

    def body(k2, carry):
        ka = 2 * k2

        @pl.when(ka > 0)
        def _():
            wait_idx(0)

        process(ka, 0)
        wait_idx(1)
        process(ka + 1, 1)
        return carry

    lax.fori_loop(0, NSEGE // 2, body, 0)

    pltpu.sync_copy(acc_v, out_hbm.at[pl.ds(wid * CW, CW)])


EPT2 = EPAD // NW


@functools.partial(
    pl.kernel,
    mesh=_mesh,
    compiler_params=pltpu.CompilerParams(needs_layout_passes=False),
    out_type=jax.ShapeDtypeStruct((NW, NPAD), jnp.float32),
    scratch_types=[
        pltpu.VMEM((NPAD,), jnp.float32),
        pltpu.VMEM((EPT2,), jnp.int32),
    ],
)
def _cnt_call(dst_hbm, out_hbm, cnt_v, dst_v):
    c = lax.axis_index("c")
    s = lax.axis_index("s")
    wid = c * TILES + s

    zero16 = jnp.zeros((16,), jnp.float32)

    def zbody(i, carry):
        cnt_v[pl.ds(i * 16, 16)] = zero16
        return carry

    lax.fori_loop(0, NPAD // 16, zbody, 0)

    pltpu.sync_copy(dst_hbm.at[pl.ds(wid * EPT2, EPT2)], dst_v)
    ones16 = jnp.full((16,), 1.0, jnp.float32)

    def blk(b, carry):
        d16 = dst_v[pl.ds(b * 16, 16)]
        plsc.addupdate_scatter(cnt_v, [d16], ones16)
        return carry

    lax.fori_loop(0, EPT2 // 16, blk, 0)
    pltpu.sync_copy(cnt_v, out_hbm.at[wid])


BR = 1280
DEGW = 16


def _tc1_body(xt_ref, w1t_ref, cnt_ref, y_ref, d_ref):
    cnt = jnp.sum(cnt_ref[...], axis=0, keepdims=True)
    d = lax.rsqrt(cnt + 1.0)
    y_ref[...] = jnp.dot(w1t_ref[...], xt_ref[...],
                         preferred_element_type=jnp.float32) * d
    d_ref[...] = jnp.broadcast_to(d, (DEGW, BR))


def _tc2_body(acc_ref, y1_ref, d_ref, b1_ref, w2t_ref, y2_ref):
    d = d_ref[0:1, :]
    h = jnp.maximum((acc_ref[...] + y1_ref[...]) * d + b1_ref[:, 0:1], 0.0)
    y2_ref[...] = jnp.dot(w2t_ref[...], h,
                          preferred_element_type=jnp.float32) * d


def _tc3_body(acc_ref, y2_ref, d_ref, b2_ref, o_ref):
    d = d_ref[0:1, :]
    o_ref[...] = (acc_ref[...] + y2_ref[...]) * d + b2_ref[:, 0:1]


_tc1 = pl.pallas_call(
    _tc1_body,
    grid=(NPAD // BR,),
    in_specs=[
        pl.BlockSpec((D, BR), lambda i: (0, i)),
        pl.BlockSpec((D, D), lambda i: (0, 0)),
        pl.BlockSpec((NW, BR), lambda i: (0, i)),
    ],
    out_specs=[
        pl.BlockSpec((D, BR), lambda i: (0, i)),
        pl.BlockSpec((DEGW, BR), lambda i: (0, i)),
    ],
    out_shape=[
        jax.ShapeDtypeStruct((D, NPAD), jnp.float32),
        jax.ShapeDtypeStruct((DEGW, NPAD), jnp.float32),
    ],
)

_tc2 = pl.pallas_call(
    _tc2_body,
    grid=(NPAD // BR,),
    in_specs=[
        pl.BlockSpec((D, BR), lambda i: (0, i)),
        pl.BlockSpec((D, BR), lambda i: (0, i)),
        pl.BlockSpec((DEGW, BR), lambda i: (0, i)),
        pl.BlockSpec((D, D), lambda i: (0, 0)),
        pl.BlockSpec((D, D), lambda i: (0, 0)),
    ],
    out_specs=pl.BlockSpec((D, BR), lambda i: (0, i)),
    out_shape=jax.ShapeDtypeStruct((D, NPAD), jnp.float32),
)

_tc3 = pl.pallas_call(
    _tc3_body,
    grid=(NPAD // BR,),
    in_specs=[
        pl.BlockSpec((D, BR), lambda i: (0, i)),
        pl.BlockSpec((D, BR), lambda i: (0, i)),
        pl.BlockSpec((DEGW, BR), lambda i: (0, i)),
        pl.BlockSpec((D, D), lambda i: (0, 0)),
    ],
    out_specs=pl.BlockSpec((D, BR), lambda i: (0, i)),
    out_shape=jax.ShapeDtypeStruct((D, NPAD), jnp.float32),
)


@jax.jit
def kernel(x, edge_index, W1, b1, W2, b2):
    src = edge_index[0].astype(jnp.int32)
    dst = edge_index[1].astype(jnp.int32)
    pad = jnp.full((EPAD - N_EDGES,), N_NODES, jnp.int32)
    srcp = jnp.concatenate([src, pad])
    dstp = jnp.concatenate([dst, pad])

    x_pad = jnp.zeros((NPAD, D), jnp.float32).at[:N_NODES].set(x)
    xt = x_pad.T
    b1m = jnp.broadcast_to(b1.reshape(D, 1), (D, D))
    b2m = jnp.broadcast_to(b2.reshape(D, 1), (D, D))
    cnt = _cnt_call(dstp)
    y1, dmat = _tc1(xt, W1.T, cnt)
    acc1 = _msg_call(y1.reshape(-1), srcp, dstp).reshape(D, NPAD)
    y2 = _tc2(acc1, y1, dmat, b1m, W2.T)
    acc2 = _msg_call(y2.reshape(-1), srcp, dstp).reshape(D, NPAD)
    out = _tc3(acc2, y2, dmat, b2m)
    return out.T[:N_NODES]


# speedup vs baseline: 1.4532x; 1.0040x over previous
import functools

import jax
import jax.numpy as jnp
from jax import lax
from jax.experimental import pallas as pl
from jax.experimental.pallas import tpu as pltpu
from jax.experimental.pallas import tpu_sc as plsc

N_NODES = 10000
D = 128
N_EDGES = 320000

NPAD = 10240
NW = 32
TILES = 16
CPT = D // NW
SEGE = 4096
EPAD = 327680
NSEGE = EPAD // SEGE
CW = CPT * NPAD

_mesh = plsc.VectorSubcoreMesh(core_axis_name="c", subcore_axis_name="s")


@functools.partial(
    pl.kernel,
    mesh=_mesh,
    compiler_params=pltpu.CompilerParams(needs_layout_passes=False),
    out_type=jax.ShapeDtypeStruct((D * NPAD,), jnp.float32),
    scratch_types=[
        pltpu.VMEM((CW,), jnp.float32),
        pltpu.VMEM((CW,), jnp.float32),
        pltpu.VMEM((2, SEGE), jnp.int32),
        pltpu.VMEM((2, SEGE), jnp.int32),
        pltpu.SemaphoreType.DMA,
        pltpu.SemaphoreType.DMA,
        pltpu.SemaphoreType.DMA,
        pltpu.SemaphoreType.DMA,
    ],
)
def _msg_call(y_hbm, src_hbm, dst_hbm, out_hbm,
              y_v, acc_v, src_v, dst_v, sems0, sems1, semd0, semd1):
    semss = [sems0, sems1]
    semds = [semd0, semd1]
    c = lax.axis_index("c")
    s = lax.axis_index("s")
    wid = c * TILES + s

    zero16 = jnp.zeros((16,), jnp.float32)

    def zbody(i, carry):
        acc_v[pl.ds(i * 16, 16)] = zero16
        return carry

    lax.fori_loop(0, CW // 16, zbody, 0)

    pltpu.sync_copy(y_hbm.at[pl.ds(wid * CW, CW)], y_v)

    pltpu.sync_copy(src_hbm.at[pl.ds(0, SEGE)], src_v.at[0])
    pltpu.sync_copy(dst_hbm.at[pl.ds(0, SEGE)], dst_v.at[0])
    pltpu.async_copy(src_hbm.at[pl.ds(SEGE, SEGE)], src_v.at[1], sems1)
    pltpu.async_copy(dst_hbm.at[pl.ds(SEGE, SEGE)], dst_v.at[1], semd1)

    colbase = [jnp.full((16,), kk * NPAD, jnp.int32) for kk in range(CPT)]

    def process(k, slot):
        def blk(b, carry):
            for g in range(2):
                s16 = src_v[slot, pl.ds(b * 32 + g * 16, 16)]
                d16 = dst_v[slot, pl.ds(b * 32 + g * 16, 16)]
                for kk in range(CPT):
                    v = plsc.load_gather(y_v, [s16 + colbase[kk]])
                    plsc.addupdate_scatter(acc_v, [d16 + colbase[kk]], v)
            return carry

        lax.fori_loop(0, SEGE // 32, blk, 0)

        @pl.when(k + 2 < NSEGE)
        def _():
            pltpu.async_copy(src_hbm.at[pl.ds((k + 2) * SEGE, SEGE)],
                             src_v.at[slot], semss[slot])
            pltpu.async_copy(dst_hbm.at[pl.ds((k + 2) * SEGE, SEGE)],
                             dst_v.at[slot], semds[slot])

    def wait_idx(slot):
        pltpu.make_async_copy(src_hbm.at[pl.ds(0, SEGE)],
                              src_v.at[slot], semss[slot]).wait()
        pltpu.make_async_copy(dst_hbm.at[pl.ds(0, SEGE)],
                              dst_v.at[slot], semds[slot]).wait()

    def body(k2, carry):
        ka = 2 * k2

        @pl.when(ka > 0)
        def _():
            wait_idx(0)

        process(ka, 0)
        wait_idx(1)
        process(ka + 1, 1)
        return carry

    lax.fori_loop(0, NSEGE // 2, body, 0)

    pltpu.sync_copy(acc_v, out_hbm.at[pl.ds(wid * CW, CW)])


EPT2 = EPAD // NW


@functools.partial(
    pl.kernel,
    mesh=_mesh,
    compiler_params=pltpu.CompilerParams(needs_layout_passes=False),
    out_type=jax.ShapeDtypeStruct((NW, NPAD), jnp.float32),
    scratch_types=[
        pltpu.VMEM((NPAD,), jnp.float32),
        pltpu.VMEM((EPT2,), jnp.int32),
    ],
)
def _cnt_call(dst_hbm, out_hbm, cnt_v, dst_v):
    c = lax.axis_index("c")
    s = lax.axis_index("s")
    wid = c * TILES + s

    zero16 = jnp.zeros((16,), jnp.float32)

    def zbody(i, carry):
        cnt_v[pl.ds(i * 16, 16)] = zero16
        return carry

    lax.fori_loop(0, NPAD // 16, zbody, 0)

    pltpu.sync_copy(dst_hbm.at[pl.ds(wid * EPT2, EPT2)], dst_v)
    ones16 = jnp.full((16,), 1.0, jnp.float32)

    def blk(b, carry):
        d16 = dst_v[pl.ds(b * 16, 16)]
        plsc.addupdate_scatter(cnt_v, [d16], ones16)
        return carry

    lax.fori_loop(0, EPT2 // 16, blk, 0)
    pltpu.sync_copy(cnt_v, out_hbm.at[wid])


BR = 1280
DEGW = 16


def _tc1_body(xt_ref, w1t_ref, cnt_ref, y_ref, d_ref):
    cnt = jnp.sum(cnt_ref[...], axis=0, keepdims=True)
    d = lax.rsqrt(cnt + 1.0)
    y_ref[...] = jnp.dot(w1t_ref[...], xt_ref[...],
                         preferred_element_type=jnp.float32) * d
    d_ref[...] = jnp.broadcast_to(d, (DEGW, BR))


def _tc2_body(acc_ref, y1_ref, d_ref, b1_ref, w2t_ref, y2_ref):
    d = d_ref[0:1, :]
    h = jnp.maximum((acc_ref[...] + y1_ref[...]) * d + b1_ref[:, 0:1], 0.0)
    y2_ref[...] = jnp.dot(w2t_ref[...], h,
                          preferred_element_type=jnp.float32) * d


def _tc3_body(acc_ref, y2_ref, d_ref, b2_ref, o_ref):
    d = d_ref[0:1, :]
    o_ref[...] = (acc_ref[...] + y2_ref[...]) * d + b2_ref[:, 0:1]


_tc1 = pl.pallas_call(
    _tc1_body,
    grid=(NPAD // BR,),
    in_specs=[
        pl.BlockSpec((D, BR), lambda i: (0, i)),
        pl.BlockSpec((D, D), lambda i: (0, 0)),
        pl.BlockSpec((NW, BR), lambda i: (0, i)),
    ],
    out_specs=[
        pl.BlockSpec((D, BR), lambda i: (0, i)),
        pl.BlockSpec((DEGW, BR), lambda i: (0, i)),
    ],
    out_shape=[
        jax.ShapeDtypeStruct((D, NPAD), jnp.float32),
        jax.ShapeDtypeStruct((DEGW, NPAD), jnp.float32),
    ],
)

_tc2 = pl.pallas_call(
    _tc2_body,
    grid=(NPAD // BR,),
    in_specs=[
        pl.BlockSpec((D, BR), lambda i: (0, i)),
        pl.BlockSpec((D, BR), lambda i: (0, i)),
        pl.BlockSpec((DEGW, BR), lambda i: (0, i)),
        pl.BlockSpec((D, D), lambda i: (0, 0)),
        pl.BlockSpec((D, D), lambda i: (0, 0)),
    ],
    out_specs=pl.BlockSpec((D, BR), lambda i: (0, i)),
    out_shape=jax.ShapeDtypeStruct((D, NPAD), jnp.float32),
)

_tc3 = pl.pallas_call(
    _tc3_body,
    grid=(NPAD // BR,),
    in_specs=[
        pl.BlockSpec((D, BR), lambda i: (0, i)),
        pl.BlockSpec((D, BR), lambda i: (0, i)),
        pl.BlockSpec((DEGW, BR), lambda i: (0, i)),
        pl.BlockSpec((D, D), lambda i: (0, 0)),
    ],
    out_specs=pl.BlockSpec((D, BR), lambda i: (0, i)),
    out_shape=jax.ShapeDtypeStruct((D, NPAD), jnp.float32),
)


@jax.jit
def kernel(x, edge_index, W1, b1, W2, b2):
    src = edge_index[0].astype(jnp.int32)
    dst = edge_index[1].astype(jnp.int32)
    pad = jnp.full((EPAD - N_EDGES,), N_NODES, jnp.int32)
    srcp = jnp.concatenate([src, pad])
    dstp = jnp.concatenate([dst, pad])

    x_pad = jnp.zeros((NPAD, D), jnp.float32).at[:N_NODES].set(x)
    xt = x_pad.T
    b1m = jnp.broadcast_to(b1.reshape(D, 1), (D, D))
    b2m = jnp.broadcast_to(b2.reshape(D, 1), (D, D))
    cnt = _cnt_call(dstp)
    y1, dmat = _tc1(xt, W1.T, cnt)
    acc1 = _msg_call(y1.reshape(-1), srcp, dstp).reshape(D, NPAD)
    y2 = _tc2(acc1, y1, dmat, b1m, W2.T)
    acc2 = _msg_call(y2.reshape(-1), srcp, dstp).reshape(D, NPAD)
    out = _tc3(acc2, y2, dmat, b2m)
    return out.T[:N_NODES]
